# baseline (speedup 1.0000x reference)
import functools

import jax
import jax.numpy as jnp
from jax import lax
from jax.experimental import pallas as pl
from jax.experimental.pallas import tpu as pltpu
from jax.experimental.pallas import tpu_sc as plsc


def _rank_body(seg_len, chunk, s_ref, nidxs_ref, bg_ref, nbr_ref):
    seg = pl.program_id(0)
    offset = seg * seg_len
    nch = seg_len // chunk
    sB = s_ref[...]
    sT = jnp.transpose(sB, (1, 0))
    nbrT = jnp.transpose(nidxs_ref[...], (1, 0))
    for a in range(nch):
        nbr_ref[a:a + 1, :] = nbrT[1:2, a * chunk:(a + 1) * chunk]
    r_iota = lax.broadcasted_iota(jnp.int32, (chunk, chunk), 0)
    lane = lax.broadcasted_iota(jnp.int32, (chunk, chunk), 1)
    for a in range(nch):
        s_i = sB[a:a + 1, :]
        acc = jnp.zeros((1, chunk), jnp.int32)
        for b in range(nch):
            s_j = sT[:, b:b + 1]
            jidx = r_iota + (b * chunk)
            iidx = lane + (a * chunk)
            before = (s_j > s_i) | ((s_j == s_i) & (jidx < iidx))
            acc = acc + jnp.sum(before.astype(jnp.int32), axis=0,
                                keepdims=True)
        bg_ref[a:a + 1, :] = acc + offset


def _tc_rank(s, nidxs, num_seg, chunk):
    n = s.shape[0]
    seg_len = n // num_seg
    nch = seg_len // chunk
    body = functools.partial(_rank_body, seg_len, chunk)
    s28 = s[:, 0].reshape(num_seg * nch, chunk)
    return pl.pallas_call(
        body,
        grid=(num_seg,),
        in_specs=[
            pl.BlockSpec((nch, chunk), lambda k: (k, 0)),
            pl.BlockSpec((seg_len, nidxs.shape[1]), lambda k: (k, 0)),
        ],
        out_specs=[
            pl.BlockSpec((nch, chunk), lambda k: (k, 0)),
            pl.BlockSpec((nch, chunk), lambda k: (k, 0)),
        ],
        out_shape=[
            jax.ShapeDtypeStruct((num_seg * nch, chunk), jnp.int32),
            jax.ShapeDtypeStruct((num_seg * nch, chunk), jnp.int32),
        ],
    )(s28, nidxs)


def _make_sc_scatter(n, f, seg_len, n_workers, chunk):
    rows_per_w = n // n_workers
    nch = rows_per_w // chunk
    tiles_per_seg = seg_len // rows_per_w
    mesh = plsc.VectorSubcoreMesh(core_axis_name="c", subcore_axis_name="s")
    nc = mesh.num_cores

    @functools.partial(
        pl.kernel,
        out_type=jax.ShapeDtypeStruct((n, 2 * f), jnp.float32),
        mesh=mesh,
        scratch_types=[
            pltpu.VMEM((nch, chunk), jnp.int32),
            pltpu.VMEM((nch, chunk), jnp.int32),
            pltpu.VMEM((3, chunk, f), jnp.float32),
        ] + [pltpu.SemaphoreType.DMA] * 8,
    )
    def sc_scatter(features_hbm, nbr_hbm, bgr_hbm, out_hbm,
                   bg_v, nbr_v, ring,
                   sem_ib, sem_in, g0, g1, g2, s0, s1, s2):
        wid = lax.axis_index("s") * nc + lax.axis_index("c")
        base = wid * rows_per_w
        gsem = (g0, g1, g2)
        ssem = (s0, s1, s2)

        cps_bg = [pltpu.async_copy(
            bgr_hbm.at[pl.ds((base + t * chunk) // 256, 1),
                       pl.ds((t * chunk) % 256, chunk)],
            bg_v.at[pl.ds(t, 1)], sem_ib) for t in range(nch)]
        cps_nb = [pltpu.async_copy(
            nbr_hbm.at[pl.ds((base + t * chunk) // 256, 1),
                       pl.ds((t * chunk) % 256, chunk)],
            nbr_v.at[pl.ds(t, 1)], sem_in) for t in range(nch)]
        for cp in cps_nb:
            cp.wait()

        ntr = 2 * nch
        g_d = [None] * ntr
        s_d = [None] * ntr

        def fire_gather(k):
            b = k % 3
            t = k >> 1
            if k & 1:
                src = features_hbm.at[nbr_v.at[t]]
            else:
                src = features_hbm.at[pl.ds(base + t * chunk, chunk)]
            g_d[k] = pltpu.async_copy(src, ring.at[b], gsem[b])

        def fire_scatter(k):
            b = k % 3
            t = k >> 1
            half = f if (k & 1) else 0
            s_d[k] = pltpu.async_copy(
                ring.at[b], out_hbm.at[bg_v.at[t], pl.ds(half, f)], ssem[b])

        fire_gather(0)
        fire_gather(1)
        fire_gather(2)
        for cp in cps_bg:
            cp.wait()
        for k in range(ntr):
            g_d[k].wait()
            fire_scatter(k)
            if k + 3 < ntr:
                s_d[k].wait()
                fire_gather(k + 3)
        for k in range(max(0, ntr - 3), ntr):
            s_d[k].wait()

    return sc_scatter


def kernel(features, score, distances, nidxs, row_splits, tidxs):
    n, f = features.shape
    num_seg = row_splits.shape[0] - 1
    seg_len = n // num_seg

    s = jax.nn.sigmoid(score)
    bg64, nbr64 = _tc_rank(s, nidxs, num_seg, 256)
    backgather = bg64.reshape(n, 1)

    chunk = 128
    sc = _make_sc_scatter(n, f, seg_len, n_workers=32, chunk=chunk)
    out_features = sc(features, nbr64, bg64)

    return out_features, row_splits, backgather


# speedup vs baseline: 1.1323x; 1.1323x over previous
import functools

import jax
import jax.numpy as jnp
from jax import lax
from jax.experimental import pallas as pl
from jax.experimental.pallas import tpu as pltpu
from jax.experimental.pallas import tpu_sc as plsc


def _rank_body(seg_len, chunk, s_ref, bg_ref):
    seg = pl.program_id(0)
    offset = seg * seg_len
    nch = seg_len // chunk
    sB = s_ref[...]
    sT = jnp.transpose(sB, (1, 0))
    r_iota = lax.broadcasted_iota(jnp.int32, (chunk, chunk), 0)
    lane = lax.broadcasted_iota(jnp.int32, (chunk, chunk), 1)
    accs = [jnp.zeros((1, chunk), jnp.int32) for _ in range(nch)]
    for a in range(nch):
        s_i = sB[a:a + 1, :]
        s_j = sT[:, a:a + 1]
        before = (s_j > s_i) | ((s_j == s_i) & (r_iota < lane))
        accs[a] = accs[a] + jnp.sum(before.astype(jnp.int32), axis=0,
                                    keepdims=True)
    for a in range(nch):
        for b in range(a + 1, nch):
            hi_beats_lo = sT[:, b:b + 1] > sB[a:a + 1, :]
            lo_beats_hi = sT[:, a:a + 1] >= sB[b:b + 1, :]
            accs[a] = accs[a] + jnp.sum(hi_beats_lo.astype(jnp.int32),
                                        axis=0, keepdims=True)
            accs[b] = accs[b] + jnp.sum(lo_beats_hi.astype(jnp.int32),
                                        axis=0, keepdims=True)
    for a in range(nch):
        bg_ref[a:a + 1, :] = accs[a] + offset


def _tc_rank(s, num_seg, chunk):
    n = s.shape[0]
    seg_len = n // num_seg
    nch = seg_len // chunk
    body = functools.partial(_rank_body, seg_len, chunk)
    s28 = s[:, 0].reshape(num_seg * nch, chunk)
    return pl.pallas_call(
        body,
        grid=(num_seg,),
        in_specs=[pl.BlockSpec((nch, chunk), lambda k: (k, 0))],
        out_specs=pl.BlockSpec((nch, chunk), lambda k: (k, 0)),
        out_shape=jax.ShapeDtypeStruct((num_seg * nch, chunk), jnp.int32),
    )(s28)


def _make_sc_scatter(n, f, seg_len, n_workers, chunk):
    rows_per_w = n // n_workers
    nch = rows_per_w // chunk
    tiles_per_seg = seg_len // rows_per_w
    mesh = plsc.VectorSubcoreMesh(core_axis_name="c", subcore_axis_name="s")
    nc = mesh.num_cores

    @functools.partial(
        pl.kernel,
        out_type=jax.ShapeDtypeStruct((n, 2 * f), jnp.float32),
        mesh=mesh,
        scratch_types=[
            pltpu.VMEM((nch, chunk), jnp.int32),
            pltpu.VMEM((nch, chunk), jnp.int32),
            pltpu.VMEM((3, chunk, f), jnp.float32),
        ] + [pltpu.SemaphoreType.DMA] * 8,
    )
    def sc_scatter(features_hbm, nbr_hbm, bgr_hbm, out_hbm,
                   bg_v, nbr_v, ring,
                   sem_ib, sem_in, g0, g1, g2, s0, s1, s2):
        wid = lax.axis_index("s") * nc + lax.axis_index("c")
        base = wid * rows_per_w
        gsem = (g0, g1, g2)
        ssem = (s0, s1, s2)

        cps_bg = [pltpu.async_copy(
            bgr_hbm.at[pl.ds((base + t * chunk) // 256, 1),
                       pl.ds((t * chunk) % 256, chunk)],
            bg_v.at[pl.ds(t, 1)], sem_ib) for t in range(nch)]
        cp_nb = pltpu.async_copy(
            nbr_hbm.at[pl.ds(wid * nch, nch)], nbr_v, sem_in)
        cp_nb.wait()

        ntr = 2 * nch
        g_d = [None] * ntr
        s_d = [None] * ntr

        def fire_gather(k):
            b = k % 3
            t = k >> 1
            if k & 1:
                src = features_hbm.at[nbr_v.at[t]]
            else:
                src = features_hbm.at[pl.ds(base + t * chunk, chunk)]
            g_d[k] = pltpu.async_copy(src, ring.at[b], gsem[b])

        def fire_scatter(k):
            b = k % 3
            t = k >> 1
            half = f if (k & 1) else 0
            s_d[k] = pltpu.async_copy(
                ring.at[b], out_hbm.at[bg_v.at[t], pl.ds(half, f)], ssem[b])

        fire_gather(0)
        fire_gather(1)
        fire_gather(2)
        for cp in cps_bg:
            cp.wait()
        for k in range(ntr):
            g_d[k].wait()
            fire_scatter(k)
            if k + 3 < ntr:
                s_d[k].wait()
                fire_gather(k + 3)
        for k in range(max(0, ntr - 3), ntr):
            s_d[k].wait()

    return sc_scatter


def kernel(features, score, distances, nidxs, row_splits, tidxs):
    n, f = features.shape
    num_seg = row_splits.shape[0] - 1
    seg_len = n // num_seg

    s = jax.nn.sigmoid(score)
    bg64 = _tc_rank(s, num_seg, 256)
    backgather = bg64.reshape(n, 1)

    chunk = 128
    nbr_c = nidxs[:, 1].reshape(n // chunk, chunk)
    sc = _make_sc_scatter(n, f, seg_len, n_workers=32, chunk=chunk)
    out_features = sc(features, nbr_c, bg64)

    return out_features, row_splits, backgather


# speedup vs baseline: 1.1328x; 1.0004x over previous
import functools

import jax
import jax.numpy as jnp
from jax import lax
from jax.experimental import pallas as pl
from jax.experimental.pallas import tpu as pltpu
from jax.experimental.pallas import tpu_sc as plsc


def _rank_body(seg_len, chunk, s_ref, bg_ref):
    seg = pl.program_id(0)
    offset = seg * seg_len
    nch = seg_len // chunk
    sB = s_ref[...]
    sT = jnp.transpose(sB, (1, 0))
    r_iota = lax.broadcasted_iota(jnp.int32, (chunk, chunk), 0)
    lane = lax.broadcasted_iota(jnp.int32, (chunk, chunk), 1)
    accs = [jnp.zeros((1, chunk), jnp.int32) for _ in range(nch)]
    for a in range(nch):
        s_i = sB[a:a + 1, :]
        s_j = sT[:, a:a + 1]
        before = (s_j > s_i) | ((s_j == s_i) & (r_iota < lane))
        accs[a] = accs[a] + jnp.sum(before.astype(jnp.int32), axis=0,
                                    keepdims=True)
    for a in range(nch):
        for b in range(a + 1, nch):
            hi_beats_lo = sT[:, b:b + 1] > sB[a:a + 1, :]
            lo_beats_hi = sT[:, a:a + 1] >= sB[b:b + 1, :]
            accs[a] = accs[a] + jnp.sum(hi_beats_lo.astype(jnp.int32),
                                        axis=0, keepdims=True)
            accs[b] = accs[b] + jnp.sum(lo_beats_hi.astype(jnp.int32),
                                        axis=0, keepdims=True)
    for a in range(nch):
        bg_ref[a:a + 1, :] = accs[a] + offset


def _tc_rank(s, num_seg, chunk):
    n = s.shape[0]
    seg_len = n // num_seg
    nch = seg_len // chunk
    body = functools.partial(_rank_body, seg_len, chunk)
    s28 = s[:, 0].reshape(num_seg * nch, chunk)
    return pl.pallas_call(
        body,
        grid=(num_seg,),
        in_specs=[pl.BlockSpec((nch, chunk), lambda k: (k, 0))],
        out_specs=pl.BlockSpec((nch, chunk), lambda k: (k, 0)),
        out_shape=jax.ShapeDtypeStruct((num_seg * nch, chunk), jnp.int32),
    )(s28)


def _make_sc_scatter(n, f, n_workers, chunk):
    rows_per_w = n // n_workers
    nch = rows_per_w // chunk
    mesh = plsc.VectorSubcoreMesh(core_axis_name="c", subcore_axis_name="s")
    nc = mesh.num_cores

    @functools.partial(
        pl.kernel,
        out_type=jax.ShapeDtypeStruct((n, 2 * f), jnp.float32),
        mesh=mesh,
        scratch_types=[
            pltpu.VMEM((nch, chunk), jnp.int32),
            pltpu.VMEM((nch, chunk), jnp.int32),
            pltpu.VMEM((3, chunk, f), jnp.float32),
        ] + [pltpu.SemaphoreType.DMA] * 8,
    )
    def sc_scatter(features_hbm, nbr_hbm, bgr_hbm, out_hbm,
                   bg_v, nbr_v, ring,
                   sem_ib, sem_in, g0, g1, g2, s0, s1, s2):
        wid = lax.axis_index("s") * nc + lax.axis_index("c")
        base = wid * rows_per_w
        gsem = (g0, g1, g2)
        ssem = (s0, s1, s2)

        cps_bg = [pltpu.async_copy(
            bgr_hbm.at[pl.ds((base + t * chunk) // 256, 1),
                       pl.ds((t * chunk) % 256, chunk)],
            bg_v.at[pl.ds(t, 1)], sem_ib) for t in range(nch)]
        cp_nb = pltpu.async_copy(
            nbr_hbm.at[pl.ds(wid * nch, nch)], nbr_v, sem_in)
        cp_nb.wait()

        ntr = 2 * nch
        g_d = [None] * ntr
        s_d = [None] * ntr

        def fire_gather(k):
            b = k % 3
            t = k >> 1
            if k & 1:
                src = features_hbm.at[nbr_v.at[t]]
            else:
                src = features_hbm.at[pl.ds(base + t * chunk, chunk)]
            g_d[k] = pltpu.async_copy(src, ring.at[b], gsem[b])

        def fire_scatter(k):
            b = k % 3
            t = k >> 1
            half = f if (k & 1) else 0
            s_d[k] = pltpu.async_copy(
                ring.at[b], out_hbm.at[bg_v.at[t], pl.ds(half, f)], ssem[b])

        fire_gather(0)
        fire_gather(1)
        fire_gather(2)
        for cp in cps_bg:
            cp.wait()
        for k in range(ntr):
            g_d[k].wait()
            fire_scatter(k)
            if k + 3 < ntr:
                s_d[k].wait()
                fire_gather(k + 3)
        for k in range(max(0, ntr - 3), ntr):
            s_d[k].wait()

    return sc_scatter


def kernel(features, score, distances, nidxs, row_splits, tidxs):
    n, f = features.shape
    num_seg = row_splits.shape[0] - 1

    s = jax.nn.sigmoid(score)
    bg64 = _tc_rank(s, num_seg, 256)
    backgather = bg64.reshape(n, 1)

    chunk = 128
    nbr_c = nidxs[:, 1].reshape(n // chunk, chunk)
    sc = _make_sc_scatter(n, f, n_workers=32, chunk=chunk)
    out_features = sc(features, nbr_c, bg64)

    return out_features, row_splits, backgather
